# baseline (speedup 1.0000x reference)
import functools

import jax
import jax.numpy as jnp
from jax.experimental import pallas as pl
from jax.experimental.pallas import tpu as pltpu

UNITS = 4096
K = 409
INPUT_DIM = 256
BATCH_BLOCK = 512
N_SEARCH_ITERS = 31


def _wta_kernel(x_ref, w_ref, b_ref, o_ref):
    x = x_ref[...]
    w = w_ref[...]
    b = b_ref[...]
    out = jnp.dot(x, w, preferred_element_type=jnp.float32) + b
    out = jnp.maximum(out, 0.0)

    bits = jax.lax.bitcast_convert_type(out, jnp.int32)

    bb = out.shape[0]
    lo = jnp.zeros((bb, 1), jnp.int32)
    hi = jnp.max(bits, axis=1, keepdims=True)

    def body(_, carry):
        lo, hi = carry
        mid = lo + ((hi - lo + 1) >> 1)
        cnt = jnp.sum((bits >= mid).astype(jnp.int32), axis=1, keepdims=True)
        ge = cnt >= K
        lo = jnp.where(ge, mid, lo)
        hi = jnp.where(ge, hi, mid - 1)
        return lo, hi

    lo, _ = jax.lax.fori_loop(0, N_SEARCH_ITERS, body, (lo, hi))
    o_ref[...] = jnp.where(bits >= lo, out, 0.0)


@jax.jit
def kernel(inputs, W, b):
    batch = inputs.shape[0]
    grid = (batch // BATCH_BLOCK,)
    b2 = b.reshape(1, UNITS)
    return pl.pallas_call(
        _wta_kernel,
        grid=grid,
        in_specs=[
            pl.BlockSpec((BATCH_BLOCK, INPUT_DIM), lambda i: (i, 0)),
            pl.BlockSpec((INPUT_DIM, UNITS), lambda i: (0, 0)),
            pl.BlockSpec((1, UNITS), lambda i: (0, 0)),
        ],
        out_specs=pl.BlockSpec((BATCH_BLOCK, UNITS), lambda i: (i, 0)),
        out_shape=jax.ShapeDtypeStruct((batch, UNITS), jnp.float32),
        compiler_params=pltpu.CompilerParams(
            dimension_semantics=("parallel",),
        ),
    )(inputs, W, b2)


# speedup vs baseline: 1.3010x; 1.3010x over previous
import jax
import jax.numpy as jnp
from jax.experimental import pallas as pl
from jax.experimental.pallas import tpu as pltpu

UNITS = 4096
K = 409
INPUT_DIM = 256
BATCH_BLOCK = 512


def _count_ge(arr_s16, mid_s16):
    acc = (arr_s16 >= mid_s16).astype(jnp.int16)
    n = acc.shape[1]
    while n > 128:
        n //= 2
        acc = acc[:, :n] + acc[:, n:2 * n]
    return jnp.sum(acc.astype(jnp.int32), axis=1, keepdims=True)


def _wta_kernel(x_ref, w_ref, b_ref, o_ref):
    x = x_ref[...]
    w = w_ref[...]
    b = b_ref[...]
    out = jnp.dot(x, w, preferred_element_type=jnp.float32) + b
    out = jnp.maximum(out, 0.0)

    bits = jax.lax.bitcast_convert_type(out, jnp.int32)
    bb = out.shape[0]

    top16 = (bits >> 16).astype(jnp.int16)
    lo = jnp.zeros((bb, 1), jnp.int32)
    hi = jnp.max(bits, axis=1, keepdims=True) >> 16

    def body1(_, carry):
        lo, hi = carry
        mid = lo + ((hi - lo + 1) >> 1)
        cnt = _count_ge(top16, mid.astype(jnp.int16))
        ge = cnt >= K
        lo = jnp.where(ge, mid, lo)
        hi = jnp.where(ge, hi, mid - 1)
        return lo, hi

    lo, _ = jax.lax.fori_loop(0, 15, body1, (lo, hi))
    b16 = lo

    b16s = b16.astype(jnp.int16)
    is_b = top16 == b16s
    c_hi = _count_ge(top16, b16s + jnp.int16(1))
    k2 = K - c_hi
    low16 = ((bits & 0xFFFF) - 32768).astype(jnp.int16)
    lowm = jnp.where(is_b, low16, jnp.int16(-32768))

    lo2 = jnp.full((bb, 1), -32768, jnp.int32)
    hi2 = jnp.full((bb, 1), 32767, jnp.int32)

    def body2(_, carry):
        lo, hi = carry
        mid = lo + ((hi - lo + 1) >> 1)
        cnt = _count_ge(lowm, mid.astype(jnp.int16))
        ge = cnt >= k2
        lo = jnp.where(ge, mid, lo)
        hi = jnp.where(ge, hi, mid - 1)
        return lo, hi

    lo2, _ = jax.lax.fori_loop(0, 16, body2, (lo2, hi2))

    thr = (b16 << 16) | (lo2 + 32768)
    o_ref[...] = jnp.where(bits >= thr, out, 0.0)


@jax.jit
def kernel(inputs, W, b):
    batch = inputs.shape[0]
    grid = (batch // BATCH_BLOCK,)
    b2 = b.reshape(1, UNITS)
    return pl.pallas_call(
        _wta_kernel,
        grid=grid,
        in_specs=[
            pl.BlockSpec((BATCH_BLOCK, INPUT_DIM), lambda i: (i, 0)),
            pl.BlockSpec((INPUT_DIM, UNITS), lambda i: (0, 0)),
            pl.BlockSpec((1, UNITS), lambda i: (0, 0)),
        ],
        out_specs=pl.BlockSpec((BATCH_BLOCK, UNITS), lambda i: (i, 0)),
        out_shape=jax.ShapeDtypeStruct((batch, UNITS), jnp.float32),
        compiler_params=pltpu.CompilerParams(
            dimension_semantics=("parallel",),
        ),
    )(inputs, W, b2)
